# Initial kernel scaffold; baseline (speedup 1.0000x reference)
import jax
import jax.numpy as jnp
from jax.experimental import pallas as pl


def kernel(x, edge_index, edge_attr, le0_W, le0_b, eps0, W1_0, b1_0, g1_0, be1_0, W2_0, b2_0, og_0, ob_0, le1_W, le1_b, eps1, W1_1, b1_1, g1_1, be1_1, W2_1, b2_1, og_1, ob_1, Wf, bf):
    raise NotImplementedError("write your pallas kernel here")


# speedup vs baseline: 2.3534x; 2.3534x over previous
import functools

import jax
import jax.numpy as jnp
from jax import lax
from jax.experimental import pallas as pl
from jax.experimental.pallas import tpu as pltpu
from jax.experimental.pallas import tpu_sc as plsc

NC = 2
NS = 16
CH = 128


def _edge_embed_body(ea_ref, w_ref, b_ref, o_ref):
    dh = o_ref.shape[2]
    res = (jnp.dot(ea_ref[...], w_ref[...], preferred_element_type=jnp.float32)
           + b_ref[...])
    o_ref[0] = res[:, :dh]
    o_ref[1] = res[:, dh:]


def _edge_embed(ea, w, b, blk=4000):
    E, DE = ea.shape
    D = w.shape[1]
    return pl.pallas_call(
        _edge_embed_body,
        grid=(E // blk,),
        in_specs=[
            pl.BlockSpec((blk, DE), lambda i: (i, 0)),
            pl.BlockSpec((DE, D), lambda i: (0, 0)),
            pl.BlockSpec((1, D), lambda i: (0, 0)),
        ],
        out_specs=pl.BlockSpec((NC, blk, D // NC), lambda i: (0, i, 0)),
        out_shape=jax.ShapeDtypeStruct((NC, E, D // NC), jnp.float32),
    )(ea, w, b.reshape(1, D))


def _sc_edge_stage(x2, e2, src_m, dst_m, zeros_nd):
    _, N, Dh = x2.shape
    E = e2.shape[1]
    EPW = E // NS
    NCH = src_m.shape[1]
    RPT = ((N + NS - 1) // NS + 7) // 8 * 8
    LASTR = N - RPT * (NS - 1)
    NLSL = Dh // 16
    ACC_N = N + 8

    mesh = plsc.VectorSubcoreMesh(core_axis_name="c", subcore_axis_name="s")

    @functools.partial(
        pl.kernel,
        out_type=jax.ShapeDtypeStruct((NC, N, Dh), jnp.float32),
        mesh=mesh,
        compiler_params=pltpu.CompilerParams(use_tc_tiling_on_sc=False),
        scratch_types=[
            pltpu.VMEM((NCH, CH), jnp.int32),
            pltpu.VMEM((NCH, CH), jnp.int32),
            pltpu.VMEM((CH, Dh), jnp.float32),
            pltpu.VMEM((CH, Dh), jnp.float32),
            pltpu.VMEM_SHARED((ACC_N, Dh), jnp.float32),
            pltpu.SemaphoreType.DMA,
            pltpu.SemaphoreType.DMA,
        ],
    )
    def k(x_hbm, e_hbm, srcm_hbm, dstm_hbm, z_hbm, out_hbm,
          sidx, didx, xbuf, ebuf, acc, gsem, esem):
        c = lax.axis_index("c")
        s = lax.axis_index("s")

        @pl.when(s < NS - 1)
        def _():
            pltpu.sync_copy(z_hbm.at[pl.ds(s * RPT, RPT)],
                            acc.at[pl.ds(s * RPT, RPT)])

        @pl.when(s == NS - 1)
        def _():
            pltpu.sync_copy(z_hbm.at[pl.ds((NS - 1) * RPT, LASTR)],
                            acc.at[pl.ds((NS - 1) * RPT, LASTR)])

        pltpu.sync_copy(srcm_hbm.at[s], sidx)
        pltpu.sync_copy(dstm_hbm.at[s], didx)
        plsc.subcore_barrier()

        @pl.loop(0, NCH)
        def _chunk(j):
            eb = s * EPW + jnp.minimum(j * CH, EPW - CH)
            gcp = pltpu.async_copy(x_hbm.at[c].at[sidx.at[j]], xbuf, gsem)
            ecp = pltpu.async_copy(e_hbm.at[c].at[pl.ds(eb, CH)], ebuf, esem)
            gcp.wait()
            ecp.wait()

            @pl.loop(0, CH)
            def _row(r):
                for t in range(NLSL):
                    sl = pl.ds(t * 16, 16)
                    ebuf[r, sl] = jnp.maximum(ebuf[r, sl] + xbuf[r, sl], 0.0)

            pltpu.sync_copy(ebuf, acc.at[didx.at[j]], add=True)

        plsc.subcore_barrier()

        @pl.when(s < NS - 1)
        def _():
            pltpu.sync_copy(acc.at[pl.ds(s * RPT, RPT)],
                            out_hbm.at[c, pl.ds(s * RPT, RPT)])

        @pl.when(s == NS - 1)
        def _():
            pltpu.sync_copy(acc.at[pl.ds((NS - 1) * RPT, LASTR)],
                            out_hbm.at[c, pl.ds((NS - 1) * RPT, LASTR)])

    return k(x2, e2, src_m, dst_m, zeros_nd)


def _bn(h, g, b):
    mu = jnp.mean(h, axis=0, keepdims=True)
    var = jnp.mean((h - mu) ** 2, axis=0, keepdims=True)
    return (h - mu) * lax.rsqrt(var + 1e-5) * g + b


def _leaky(h):
    return jnp.where(h >= 0.0, h, 0.01 * h)


def _node0_body(x_ref, p_ref, w1_ref, b1_ref, g1_ref, be1_ref, w2_ref, b2_ref,
                og_ref, ob_ref, eps_ref, o_ref):
    dh = o_ref.shape[2]
    aggr = jnp.concatenate([p_ref[0], p_ref[1]], axis=1)
    z = (1.0 + eps_ref[0, 0]) * x_ref[...] + aggr
    h = jnp.dot(z, w1_ref[...], preferred_element_type=jnp.float32) + b1_ref[...]
    h = _leaky(_bn(h, g1_ref[...], be1_ref[...]))
    h = jnp.dot(h, w2_ref[...], preferred_element_type=jnp.float32) + b2_ref[...]
    h = _leaky(_bn(h, og_ref[...], ob_ref[...]))
    o_ref[0] = h[:, :dh]
    o_ref[1] = h[:, dh:]


def _node0(x, parts, w1, b1, g1, be1, w2, b2, og, ob, eps):
    N, D = x.shape
    H = w1.shape[1]
    r = lambda v: v.reshape(1, -1)
    return pl.pallas_call(
        _node0_body,
        out_shape=jax.ShapeDtypeStruct((NC, N, H // NC), jnp.float32),
    )(x, parts, w1, r(b1), r(g1), r(be1), w2, r(b2), r(og), r(ob),
      eps.reshape(1, 1))


def _node1_body(h2_ref, p_ref, w1_ref, b1_ref, g1_ref, be1_ref, w2_ref, b2_ref,
                wf_ref, bf_ref, eps_ref, o_ref):
    x = jnp.concatenate([h2_ref[0], h2_ref[1]], axis=1)
    aggr = jnp.concatenate([p_ref[0], p_ref[1]], axis=1)
    z = (1.0 + eps_ref[0, 0]) * x + aggr
    h = jnp.dot(z, w1_ref[...], preferred_element_type=jnp.float32) + b1_ref[...]
    h = _leaky(_bn(h, g1_ref[...], be1_ref[...]))
    h = jnp.dot(h, w2_ref[...], preferred_element_type=jnp.float32) + b2_ref[...]
    logits = (jnp.dot(h, wf_ref[...], preferred_element_type=jnp.float32)
              + bf_ref[...])
    m = jnp.max(logits, axis=1, keepdims=True)
    ex = jnp.exp(logits - m)
    o_ref[...] = ex / jnp.sum(ex, axis=1, keepdims=True)


def _node1(h2, parts, w1, b1, g1, be1, w2, b2, wf_pad, bf_pad, eps):
    N = h2.shape[1]
    r = lambda v: v.reshape(1, -1)
    return pl.pallas_call(
        _node1_body,
        out_shape=jax.ShapeDtypeStruct((N, wf_pad.shape[1]), jnp.float32),
    )(h2, parts, w1, r(b1), r(g1), r(be1), w2, r(b2), wf_pad, r(bf_pad),
      eps.reshape(1, 1))


def kernel(x, edge_index, edge_attr, le0_W, le0_b, eps0, W1_0, b1_0, g1_0,
           be1_0, W2_0, b2_0, og_0, ob_0, le1_W, le1_b, eps1, W1_1, b1_1,
           g1_1, be1_1, W2_1, b2_1, og_1, ob_1, Wf, bf):
    N, D = x.shape
    E = edge_index.shape[1]
    C = Wf.shape[1]
    Dh = D // NC

    src = edge_index[0]
    dst = edge_index[1]
    EPW = E // NS
    NCF = EPW // CH
    srcw = src.reshape(NS, EPW)
    dstw = dst.reshape(NS, EPW)
    if EPW % CH:
        dup = (NCF + 1) * CH - EPW
        src_m = jnp.concatenate(
            [srcw[:, :NCF * CH].reshape(NS, NCF, CH),
             srcw[:, EPW - CH:].reshape(NS, 1, CH)], axis=1)
        dst_last = jnp.where(jnp.arange(CH) < dup, N, dstw[:, EPW - CH:])
        dst_m = jnp.concatenate(
            [dstw[:, :NCF * CH].reshape(NS, NCF, CH),
             dst_last.reshape(NS, 1, CH)], axis=1)
    else:
        src_m = srcw.reshape(NS, NCF, CH)
        dst_m = dstw.reshape(NS, NCF, CH)

    x2 = jnp.stack([x[:, :Dh], x[:, Dh:]])
    zeros_nd = jnp.zeros((N, Dh), jnp.float32)

    wf_pad = jnp.zeros((Wf.shape[0], 128), jnp.float32).at[:, :C].set(Wf)
    bf_pad = jnp.full((128,), -1e30, jnp.float32).at[:C].set(bf)

    e0 = _edge_embed(edge_attr, le0_W, le0_b)
    e1 = _edge_embed(edge_attr, le1_W, le1_b)

    parts0 = _sc_edge_stage(x2, e0, src_m, dst_m, zeros_nd)
    h2 = _node0(x, parts0, W1_0, b1_0, g1_0, be1_0, W2_0, b2_0, og_0, ob_0,
                eps0)
    parts1 = _sc_edge_stage(h2, e1, src_m, dst_m, zeros_nd)
    probs = _node1(h2, parts1, W1_1, b1_1, g1_1, be1_1, W2_1, b2_1, wf_pad,
                   bf_pad, eps1)
    return probs[:, :C]
